# baseline (speedup 1.0000x reference)
import functools

import jax
import jax.numpy as jnp
from jax import lax
from jax.experimental import pallas as pl
from jax.experimental.pallas import tpu as pltpu

N = 256
E = 4096
IN = 510
OC = 32
H = 192
HOC = H * OC
OUT_SPACE = N * N - N
BE = 256
NBLK = E // BE
NCH = 4
F32 = jnp.float32
BF16 = jnp.bfloat16


def _main_body(xpc_ref, w2_ref, w1_ref, b2r_ref, wroot_ref, b1_ref,
               ea_ref, src_ref, dst_ref, bconv_ref,
               wl1_ref, bl1_ref, wl2_ref, bl2_ref, wl3_ref, bl3_ref,
               out_ref,
               acc_ref, gn_ref, w1b_ref, xb_ref, xr_ref, agg_ref, sel_ref):
    i = pl.program_id(0)

    @pl.when(i < NCH)
    def _node_phase():
        c = i
        w2c = w2_ref[...]
        lane = lax.broadcasted_iota(jnp.int32, (H, 128 * OC), 1)
        w2c = jnp.where(lane + c * 128 * OC < IN * OC, w2c, 0.0)
        w2cT = jnp.transpose(w2c.astype(BF16))
        w2c3 = w2cT.reshape(128, OC, H)
        w2cm = jnp.concatenate(
            [w2c3[:, o, :] for o in range(OC)], axis=1)
        xpc = xpc_ref[...]
        col = lax.broadcasted_iota(jnp.int32, (N, 128), 1)
        xpc = jnp.where(col + c * 128 < IN, xpc, 0.0)
        contrib = jnp.dot(xpc.astype(BF16), w2cm, preferred_element_type=F32)
        xbc = jnp.dot(xpc, b2r_ref[...], preferred_element_type=F32)
        xrc = jnp.dot(xpc, wroot_ref[...], preferred_element_type=F32)

        @pl.when(i == 0)
        def _():
            acc_ref[...] = contrib
            xb_ref[...] = xbc
            xr_ref[...] = xrc
            w1b_ref[...] = w1_ref[...].astype(BF16)
            r_idx = lax.broadcasted_iota(jnp.int32, (HOC, OC), 0) // H
            c_idx = lax.broadcasted_iota(jnp.int32, (HOC, OC), 1)
            sel_ref[...] = (r_idx == c_idx).astype(BF16)

        @pl.when(i > 0)
        def _():
            acc_ref[...] += contrib
            xb_ref[...] += xbc
            xr_ref[...] += xrc

        @pl.when(i == NCH - 1)
        def _():
            gn_ref[...] = acc_ref[...].astype(BF16)

    @pl.when(i >= NCH)
    def _edge_phase():
        j = i - NCH
        h = jnp.dot(ea_ref[...].astype(BF16), w1b_ref[...],
                    preferred_element_type=F32)
        h = jnp.maximum(h + b1_ref[...], 0.0)

        sv = src_ref[...]
        lane_n = lax.broadcasted_iota(jnp.int32, (BE, N), 1)
        P = (sv == lane_n).astype(BF16)
        Pf = (sv == lane_n).astype(F32)
        dv = dst_ref[0]
        sub_n = lax.broadcasted_iota(jnp.int32, (N, BE), 0)
        Dt = (dv == sub_n).astype(F32)

        gsrc = jnp.dot(P, gn_ref[...],
                       preferred_element_type=F32).astype(BF16)

        hb = h.astype(BF16)
        hrep = jnp.concatenate([hb] * OC, axis=1)
        prod = hrep * gsrc
        msg = jnp.dot(prod, sel_ref[...],
                      preferred_element_type=F32)
        msg = msg + jnp.dot(Pf, xb_ref[...], preferred_element_type=F32)

        part = jnp.dot(Dt, msg, preferred_element_type=F32)

        @pl.when(j == 0)
        def _():
            agg_ref[...] = part

        @pl.when(j > 0)
        def _():
            agg_ref[...] += part

        @pl.when(j == NBLK - 1)
        def _():
            conv = jnp.maximum(
                xr_ref[...] + agg_ref[...] + bconv_ref[...], 0.0)
            conv3 = conv.reshape(8, 32, OC)
            m1 = jnp.concatenate([conv3[:, t, :] for t in range(32)],
                                 axis=1)
            m2 = m1.reshape(1, 8, 1024)
            flat = jnp.concatenate([m2[:, t, :] for t in range(8)],
                                   axis=1)
            h1 = jnp.dot(flat, wl1_ref[...], preferred_element_type=F32)
            h1 = jnp.maximum(h1 + bl1_ref[...], 0.0)
            h2 = jnp.dot(h1, wl2_ref[...], preferred_element_type=F32)
            h2 = jnp.maximum(h2 + bl2_ref[...], 0.0)
            out = jnp.dot(h2, wl3_ref[...], preferred_element_type=F32)
            out_ref[...] = jnp.maximum(out + bl3_ref[...], 0.0)


def kernel(x, edge_index, batch, edge_attr, W1, b1, W2, b2, W_root, b_conv,
           W_l1, b_l1, W_l2, b_l2, W_l3, b_l3):
    del batch
    b2r = b2.reshape(IN, OC)
    src = edge_index[0].astype(jnp.int32).reshape(E, 1)
    dst3 = edge_index[1].astype(jnp.int32).reshape(NBLK, 1, BE)

    def clamp_e(i):
        return (jnp.maximum(i - NCH, 0), 0)

    out = pl.pallas_call(
        _main_body,
        grid=(NCH + NBLK,),
        in_specs=[
            pl.BlockSpec((N, 128),
                         lambda i: (0, jnp.minimum(i, NCH - 1))),
            pl.BlockSpec((H, 128 * OC),
                         lambda i: (0, jnp.minimum(i, NCH - 1))),
            pl.BlockSpec((E, H), lambda i: (0, 0)),
            pl.BlockSpec((128, OC),
                         lambda i: (jnp.minimum(i, NCH - 1), 0)),
            pl.BlockSpec((128, OC),
                         lambda i: (jnp.minimum(i, NCH - 1), 0)),
            pl.BlockSpec((1, H), lambda i: (0, 0)),
            pl.BlockSpec((BE, E), clamp_e),
            pl.BlockSpec((BE, 1), clamp_e),
            pl.BlockSpec((1, 1, BE),
                         lambda i: (jnp.maximum(i - NCH, 0), 0, 0)),
            pl.BlockSpec((1, OC), lambda i: (0, 0)),
            pl.BlockSpec((N * OC, 96), lambda i: (0, 0)),
            pl.BlockSpec((1, 96), lambda i: (0, 0)),
            pl.BlockSpec((96, 48), lambda i: (0, 0)),
            pl.BlockSpec((1, 48), lambda i: (0, 0)),
            pl.BlockSpec((48, OUT_SPACE), lambda i: (0, 0)),
            pl.BlockSpec((1, OUT_SPACE), lambda i: (0, 0)),
        ],
        out_specs=pl.BlockSpec((1, OUT_SPACE), lambda i: (0, 0)),
        out_shape=jax.ShapeDtypeStruct((1, OUT_SPACE), F32),
        scratch_shapes=[
            pltpu.VMEM((N, HOC), F32),
            pltpu.VMEM((N, HOC), BF16),
            pltpu.VMEM((E, H), BF16),
            pltpu.VMEM((N, OC), F32),
            pltpu.VMEM((N, OC), F32),
            pltpu.VMEM((N, OC), F32),
            pltpu.VMEM((HOC, OC), BF16),
        ],
    )(x, W2, W1, b2r, W_root, b1.reshape(1, H), edge_attr, src, dst3,
      b_conv.reshape(1, OC), W_l1, b_l1.reshape(1, 96), W_l2,
      b_l2.reshape(1, 48), W_l3, b_l3.reshape(1, OUT_SPACE))
    return out


# speedup vs baseline: 1.0084x; 1.0084x over previous
import functools

import jax
import jax.numpy as jnp
from jax import lax
from jax.experimental import pallas as pl
from jax.experimental.pallas import tpu as pltpu

N = 256
E = 4096
IN = 510
OC = 32
H = 192
HOC = H * OC
OUT_SPACE = N * N - N
BE = 256
NBLK = E // BE
NCH = 4
F32 = jnp.float32
BF16 = jnp.bfloat16


def _main_body(xpc_ref, w2_ref, w1_ref, b2r_ref, wroot_ref, b1_ref,
               ea_ref, src_ref, dst_ref, bconv_ref,
               wl1_ref, bl1_ref, wl2_ref, bl2_ref, wl3_ref, bl3_ref,
               out_ref,
               gn_ref, w1b_ref, xb_ref, xr_ref, agg_ref, sel_ref):
    i = pl.program_id(0)

    @pl.when(i < NCH)
    def _node_phase():
        c = i
        w2c = w2_ref[...]
        lane = lax.broadcasted_iota(jnp.int32, (H, 128 * OC), 1)
        w2c = jnp.where(lane + c * 128 * OC < IN * OC, w2c, 0.0)
        w2cT = jnp.transpose(w2c.astype(BF16))
        w2c3 = w2cT.reshape(128, OC, H)
        w2cm = jnp.concatenate(
            [w2c3[:, o, :] for o in range(OC)], axis=1)
        xpc = xpc_ref[...]
        col = lax.broadcasted_iota(jnp.int32, (N, 128), 1)
        xpc = jnp.where(col + c * 128 < IN, xpc, 0.0)
        contrib = jnp.dot(xpc.astype(BF16), w2cm, preferred_element_type=F32)
        xbc = jnp.dot(xpc, b2r_ref[...], preferred_element_type=F32)
        xrc = jnp.dot(xpc, wroot_ref[...], preferred_element_type=F32)

        @pl.when(i == 0)
        def _():
            gn_ref[...] = contrib.astype(BF16)
            xb_ref[...] = xbc
            xr_ref[...] = xrc
            w1b_ref[...] = w1_ref[...].astype(BF16)
            r_idx = lax.broadcasted_iota(jnp.int32, (HOC, OC), 0) // H
            c_idx = lax.broadcasted_iota(jnp.int32, (HOC, OC), 1)
            sel_ref[...] = (r_idx == c_idx).astype(BF16)

        @pl.when(i > 0)
        def _():
            gn_ref[...] = (gn_ref[...].astype(F32) + contrib).astype(BF16)
            xb_ref[...] += xbc
            xr_ref[...] += xrc

    @pl.when(i >= NCH)
    def _edge_phase():
        j = i - NCH
        h = jnp.dot(ea_ref[...].astype(BF16), w1b_ref[...],
                    preferred_element_type=F32)
        h = jnp.maximum(h + b1_ref[...], 0.0)

        sv = src_ref[...]
        lane_n = lax.broadcasted_iota(jnp.int32, (BE, N), 1)
        P = (sv == lane_n).astype(BF16)
        Pf = (sv == lane_n).astype(F32)
        dv = dst_ref[0]
        sub_n = lax.broadcasted_iota(jnp.int32, (N, BE), 0)
        Dt = (dv == sub_n).astype(F32)

        gsrc = jnp.dot(P, gn_ref[...],
                       preferred_element_type=F32).astype(BF16)

        hb = h.astype(BF16)
        hrep = jnp.concatenate([hb] * OC, axis=1)
        prod = hrep * gsrc
        msg = jnp.dot(prod, sel_ref[...],
                      preferred_element_type=F32)
        msg = msg + jnp.dot(Pf, xb_ref[...], preferred_element_type=F32)

        part = jnp.dot(Dt, msg, preferred_element_type=F32)

        @pl.when(j == 0)
        def _():
            agg_ref[...] = part

        @pl.when(j > 0)
        def _():
            agg_ref[...] += part

        @pl.when(j == NBLK - 1)
        def _():
            conv = jnp.maximum(
                xr_ref[...] + agg_ref[...] + bconv_ref[...], 0.0)
            conv3 = conv.reshape(8, 32, OC)
            m1 = jnp.concatenate([conv3[:, t, :] for t in range(32)],
                                 axis=1)
            m2 = m1.reshape(1, 8, 1024)
            flat = jnp.concatenate([m2[:, t, :] for t in range(8)],
                                   axis=1)
            h1 = jnp.dot(flat, wl1_ref[...], preferred_element_type=F32)
            h1 = jnp.maximum(h1 + bl1_ref[...], 0.0)
            h2 = jnp.dot(h1, wl2_ref[...], preferred_element_type=F32)
            h2 = jnp.maximum(h2 + bl2_ref[...], 0.0)
            out = jnp.dot(h2, wl3_ref[...], preferred_element_type=F32)
            out_ref[...] = jnp.maximum(out + bl3_ref[...], 0.0)


def kernel(x, edge_index, batch, edge_attr, W1, b1, W2, b2, W_root, b_conv,
           W_l1, b_l1, W_l2, b_l2, W_l3, b_l3):
    del batch
    b2r = b2.reshape(IN, OC)
    src = edge_index[0].astype(jnp.int32).reshape(E, 1)
    dst3 = edge_index[1].astype(jnp.int32).reshape(NBLK, 1, BE)

    def clamp_e(i):
        return (jnp.maximum(i - NCH, 0), 0)

    out = pl.pallas_call(
        _main_body,
        grid=(NCH + NBLK,),
        in_specs=[
            pl.BlockSpec((N, 128),
                         lambda i: (0, jnp.minimum(i, NCH - 1))),
            pl.BlockSpec((H, 128 * OC),
                         lambda i: (0, jnp.minimum(i, NCH - 1))),
            pl.BlockSpec((E, H), lambda i: (0, 0)),
            pl.BlockSpec((128, OC),
                         lambda i: (jnp.minimum(i, NCH - 1), 0)),
            pl.BlockSpec((128, OC),
                         lambda i: (jnp.minimum(i, NCH - 1), 0)),
            pl.BlockSpec((1, H), lambda i: (0, 0)),
            pl.BlockSpec((BE, E), clamp_e),
            pl.BlockSpec((BE, 1), clamp_e),
            pl.BlockSpec((1, 1, BE),
                         lambda i: (jnp.maximum(i - NCH, 0), 0, 0)),
            pl.BlockSpec((1, OC), lambda i: (0, 0)),
            pl.BlockSpec((N * OC, 96), lambda i: (0, 0)),
            pl.BlockSpec((1, 96), lambda i: (0, 0)),
            pl.BlockSpec((96, 48), lambda i: (0, 0)),
            pl.BlockSpec((1, 48), lambda i: (0, 0)),
            pl.BlockSpec((48, OUT_SPACE), lambda i: (0, 0)),
            pl.BlockSpec((1, OUT_SPACE), lambda i: (0, 0)),
        ],
        out_specs=pl.BlockSpec((1, OUT_SPACE), lambda i: (0, 0)),
        out_shape=jax.ShapeDtypeStruct((1, OUT_SPACE), F32),
        scratch_shapes=[
            pltpu.VMEM((N, HOC), BF16),
            pltpu.VMEM((E, H), BF16),
            pltpu.VMEM((N, OC), F32),
            pltpu.VMEM((N, OC), F32),
            pltpu.VMEM((N, OC), F32),
            pltpu.VMEM((HOC, OC), BF16),
        ],
    )(x, W2, W1, b2r, W_root, b1.reshape(1, H), edge_attr, src, dst3,
      b_conv.reshape(1, OC), W_l1, b_l1.reshape(1, 96), W_l2,
      b_l2.reshape(1, 48), W_l3, b_l3.reshape(1, OUT_SPACE))
    return out
